# baseline (speedup 1.0000x reference)
import functools

import jax
import jax.numpy as jnp
from jax import lax
from jax.experimental import pallas as pl
from jax.experimental.pallas import tpu as pltpu
from jax.experimental.pallas import tpu_sc as plsc

_N = 10000
_D = 128
_E = 320000
_NP = 10240
_EP = 327680
_CH = 128
_CHUNKS = _EP // (32 * _CH)
_ROWS_PER_SUB = _NP // 16
_NBUF = 2
_NSRC = 4


def _sc_agg_body(src_r, dst_r, x_r, acc_out, cnt_out,
                 srcbuf, idx_dst, rows, ones_v, zrow, zcnt,
                 acc_sp, cnt_sp, sem, sem_s, sem_z, sem_c):
    c = lax.axis_index("c")
    s = lax.axis_index("s")
    wid = c * 16 + s

    zeros16 = jnp.zeros((16,), jnp.float32)
    ones16 = jnp.ones((16,), jnp.float32)
    for r in range(16):
        for col in range(8):
            zrow[r, pl.ds(col * 16, 16)] = zeros16
    for i in range(_ROWS_PER_SUB // 16):
        zcnt[pl.ds(i * 16, 16)] = zeros16
    for i in range(_CH // 16):
        ones_v[pl.ds(i * 16, 16)] = ones16

    base = s * _ROWS_PER_SUB
    def _zero(k, carry):
        pltpu.sync_copy(zrow, acc_sp.at[pl.ds(base + k * 16, 16)])
        return carry
    lax.fori_loop(0, _ROWS_PER_SUB // 16, _zero, 0)
    pltpu.sync_copy(zcnt, cnt_sp.at[pl.ds(base, _ROWS_PER_SUB)])
    irow0 = wid * _CHUNKS
    pltpu.sync_copy(dst_r.at[pl.ds(irow0, _CHUNKS)], idx_dst)
    plsc.subcore_barrier()

    for k in range(_NSRC):
        pltpu.async_copy(src_r.at[irow0 + k], srcbuf.at[k], sem_s.at[k])
    for b in range(_NBUF):
        pltpu.make_async_copy(src_r.at[irow0 + b], srcbuf.at[b],
                              sem_s.at[b]).wait()
        pltpu.async_copy(x_r.at[srcbuf.at[b]], rows.at[b], sem.at[b])

    def _chunk(j, carry):
        slot = lax.rem(j, _NBUF)
        pltpu.make_async_copy(x_r.at[srcbuf.at[lax.rem(j, _NSRC)]],
                              rows.at[slot], sem.at[slot]).wait()
        pltpu.sync_copy(rows.at[slot], acc_sp.at[idx_dst.at[j]], add=True)
        pltpu.sync_copy(ones_v, cnt_sp.at[idx_dst.at[j]], add=True)
        @pl.when(j < _CHUNKS - _NBUF)
        def _():
            g = j + _NBUF
            gs = lax.rem(g, _NSRC)
            pltpu.make_async_copy(src_r.at[irow0 + g], srcbuf.at[gs],
                                  sem_s.at[gs]).wait()
            pltpu.async_copy(x_r.at[srcbuf.at[gs]], rows.at[slot],
                             sem.at[slot])
        @pl.when(j < _CHUNKS - _NSRC)
        def _():
            f = j + _NSRC
            pltpu.async_copy(src_r.at[irow0 + f], srcbuf.at[lax.rem(j, _NSRC)],
                             sem_s.at[lax.rem(j, _NSRC)])
        return carry
    lax.fori_loop(0, _CHUNKS, _chunk, 0)
    plsc.subcore_barrier()

    pltpu.sync_copy(acc_sp.at[pl.ds(base, _ROWS_PER_SUB)],
                    acc_out.at[c, pl.ds(base, _ROWS_PER_SUB)])
    pltpu.sync_copy(cnt_sp.at[pl.ds(base, _ROWS_PER_SUB)],
                    cnt_out.at[c, pl.ds(base, _ROWS_PER_SUB)])


_sc_agg = pl.kernel(
    _sc_agg_body,
    out_type=[jax.ShapeDtypeStruct((2, _NP, _D), jnp.float32),
              jax.ShapeDtypeStruct((2, _NP), jnp.float32)],
    mesh=plsc.VectorSubcoreMesh(core_axis_name="c", subcore_axis_name="s"),
    scratch_types=[
        pltpu.VMEM((_NSRC, _CH), jnp.int32),
        pltpu.VMEM((_CHUNKS, _CH), jnp.int32),
        pltpu.VMEM((_NBUF, _CH, _D), jnp.float32),
        pltpu.VMEM((_CH,), jnp.float32),
        pltpu.VMEM((16, _D), jnp.float32),
        pltpu.VMEM((_ROWS_PER_SUB,), jnp.float32),
        pltpu.VMEM_SHARED((_NP, _D), jnp.float32),
        pltpu.VMEM_SHARED((_NP,), jnp.float32),
        pltpu.SemaphoreType.DMA((_NBUF,)),
        pltpu.SemaphoreType.DMA((_NSRC,)),
        pltpu.SemaphoreType.DMA,
        pltpu.SemaphoreType.DMA,
    ],
)


def _combine_body(a0, a1, c0, c1, xr, wl, wr, b, o_ref, *, relu):
    cnt = jnp.maximum(c0[...] + c1[...], 1.0)
    mean = (a0[...] + a1[...]) / cnt
    r = (jnp.dot(mean, wl[...], preferred_element_type=jnp.float32)
         + jnp.dot(xr[...], wr[...], preferred_element_type=jnp.float32)
         + b[...])
    if relu:
        r = jnp.maximum(r, 0.0)
    o_ref[...] = r


def _combine(a0, a1, c0, c1, xr, wlT, wrT, b, relu):
    BR = 1000
    row = lambda i: (i, 0)
    full = lambda i: (0, 0)
    return pl.pallas_call(
        functools.partial(_combine_body, relu=relu),
        grid=(_N // BR,),
        in_specs=[
            pl.BlockSpec((BR, _D), row),
            pl.BlockSpec((BR, _D), row),
            pl.BlockSpec((BR, 1), row),
            pl.BlockSpec((BR, 1), row),
            pl.BlockSpec((BR, _D), row),
            pl.BlockSpec((_D, _D), full),
            pl.BlockSpec((_D, _D), full),
            pl.BlockSpec((1, _D), full),
        ],
        out_specs=pl.BlockSpec((BR, _D), row),
        out_shape=jax.ShapeDtypeStruct((_N, _D), jnp.float32),
    )(a0, a1, c0, c1, xr, wlT, wrT, b)


def kernel(x, edge_index, W1_l, b1_l, W1_r, W2_l, b2_l, W2_r):
    src = edge_index[0]
    dst = edge_index[1]
    pad = _EP - _E
    src_p = jnp.concatenate(
        [src, jnp.zeros((pad,), jnp.int32)]).reshape(_EP // _CH, _CH)
    dummy = _N + (jnp.arange(pad, dtype=jnp.int32) % (_NP - _N))
    dst_p = jnp.concatenate([dst, dummy]).reshape(_EP // _CH, _CH)

    acc1, cnt = _sc_agg(src_p, dst_p, x)
    c0 = cnt[0, :_N].reshape(_N, 1)
    c1 = cnt[1, :_N].reshape(_N, 1)
    h = _combine(acc1[0, :_N], acc1[1, :_N], c0, c1, x,
                 W1_l.T, W1_r.T, b1_l.reshape(1, _D), relu=True)
    acc2, _ = _sc_agg(src_p, dst_p, h)
    out = _combine(acc2[0, :_N], acc2[1, :_N], c0, c1, h,
                   W2_l.T, W2_r.T, b2_l.reshape(1, _D), relu=False)
    return out


# speedup vs baseline: 3.3634x; 3.3634x over previous
import functools

import jax
import jax.numpy as jnp
from jax import lax
from jax.experimental import pallas as pl
from jax.experimental.pallas import tpu as pltpu
from jax.experimental.pallas import tpu_sc as plsc

_N = 10000
_D = 128
_E = 320000
_NP = 10240
_EP = 327680
_CH = 128
_CHUNKS = _EP // (32 * _CH)
_ROWS_PER_SUB = _NP // 16
_NBUF = 2
_NSRC = 4


def _sc_agg_body(src_r, dst_r, x_r, acc_out, cnt_out,
                 srcbuf, idx_dst, rows, ones_v, zrow, zcnt,
                 acc_sp, cnt_sp, sem, sem_s, sem_z, sem_c):
    c = lax.axis_index("c")
    s = lax.axis_index("s")
    wid = c * 16 + s

    zeros16 = jnp.zeros((16,), jnp.float32)
    ones16 = jnp.ones((16,), jnp.float32)
    for r in range(16):
        for col in range(8):
            zrow[r, pl.ds(col * 16, 16)] = zeros16
    for i in range(_ROWS_PER_SUB // 16):
        zcnt[pl.ds(i * 16, 16)] = zeros16
    for i in range(_CH // 16):
        ones_v[pl.ds(i * 16, 16)] = ones16

    base = s * _ROWS_PER_SUB
    def _zero(k, carry):
        pltpu.sync_copy(zrow, acc_sp.at[pl.ds(base + k * 16, 16)])
        return carry
    lax.fori_loop(0, _ROWS_PER_SUB // 16, _zero, 0)
    pltpu.sync_copy(zcnt, cnt_sp.at[pl.ds(base, _ROWS_PER_SUB)])
    irow0 = wid * _CHUNKS
    pltpu.sync_copy(dst_r.at[pl.ds(irow0, _CHUNKS)], idx_dst)
    plsc.subcore_barrier()

    for k in range(_NSRC):
        pltpu.async_copy(src_r.at[irow0 + k], srcbuf.at[k], sem_s.at[k])
    for b in range(_NBUF):
        pltpu.make_async_copy(src_r.at[irow0 + b], srcbuf.at[b],
                              sem_s.at[b]).wait()
        pltpu.async_copy(x_r.at[srcbuf.at[b]], rows.at[b], sem.at[b])

    def _chunk(j, carry):
        slot = lax.rem(j, _NBUF)
        pltpu.make_async_copy(x_r.at[srcbuf.at[lax.rem(j, _NSRC)]],
                              rows.at[slot], sem.at[slot]).wait()
        pltpu.sync_copy(rows.at[slot], acc_sp.at[idx_dst.at[j]], add=True)
        pltpu.sync_copy(ones_v, cnt_sp.at[idx_dst.at[j]], add=True)
        @pl.when(j < _CHUNKS - _NBUF)
        def _():
            g = j + _NBUF
            gs = lax.rem(g, _NSRC)
            pltpu.make_async_copy(src_r.at[irow0 + g], srcbuf.at[gs],
                                  sem_s.at[gs]).wait()
            pltpu.async_copy(x_r.at[srcbuf.at[gs]], rows.at[slot],
                             sem.at[slot])
        @pl.when(j < _CHUNKS - _NSRC)
        def _():
            f = j + _NSRC
            pltpu.async_copy(src_r.at[irow0 + f], srcbuf.at[lax.rem(j, _NSRC)],
                             sem_s.at[lax.rem(j, _NSRC)])
        return carry
    lax.fori_loop(0, _CHUNKS, _chunk, 0)
    plsc.subcore_barrier()

    pltpu.sync_copy(acc_sp.at[pl.ds(base, _ROWS_PER_SUB)],
                    acc_out.at[c, pl.ds(base, _ROWS_PER_SUB)])
    pltpu.sync_copy(cnt_sp.at[pl.ds(base, _ROWS_PER_SUB)],
                    cnt_out.at[c, pl.ds(base, _ROWS_PER_SUB)])


_sc_agg = pl.kernel(
    _sc_agg_body,
    out_type=[jax.ShapeDtypeStruct((2, _NP, _D), jnp.float32),
              jax.ShapeDtypeStruct((2, _NP), jnp.float32)],
    mesh=plsc.VectorSubcoreMesh(core_axis_name="c", subcore_axis_name="s"),
    scratch_types=[
        pltpu.VMEM((_NSRC, _CH), jnp.int32),
        pltpu.VMEM((_CHUNKS, _CH), jnp.int32),
        pltpu.VMEM((_NBUF, _CH, _D), jnp.float32),
        pltpu.VMEM((_CH,), jnp.float32),
        pltpu.VMEM((16, _D), jnp.float32),
        pltpu.VMEM((_ROWS_PER_SUB,), jnp.float32),
        pltpu.VMEM_SHARED((_NP, _D), jnp.float32),
        pltpu.VMEM_SHARED((_NP,), jnp.float32),
        pltpu.SemaphoreType.DMA((_NBUF,)),
        pltpu.SemaphoreType.DMA((_NSRC,)),
        pltpu.SemaphoreType.DMA,
        pltpu.SemaphoreType.DMA,
    ],
)


def _combine_body(a0, a1, c0, c1, xr, wl, wr, b, o_ref, *, relu):
    cnt = jnp.maximum(c0[...] + c1[...], 1.0)
    mean = (a0[...] + a1[...]) / cnt
    r = (jnp.dot(mean, wl[...], preferred_element_type=jnp.float32)
         + jnp.dot(xr[...], wr[...], preferred_element_type=jnp.float32)
         + b[...])
    if relu:
        r = jnp.maximum(r, 0.0)
    o_ref[...] = r


def _combine(a0, a1, c0, c1, xr, wlT, wrT, b, relu):
    BR = 1000
    row = lambda i: (i, 0)
    full = lambda i: (0, 0)
    return pl.pallas_call(
        functools.partial(_combine_body, relu=relu),
        grid=(_N // BR,),
        in_specs=[
            pl.BlockSpec((BR, _D), row),
            pl.BlockSpec((BR, _D), row),
            pl.BlockSpec((BR, 1), row),
            pl.BlockSpec((BR, 1), row),
            pl.BlockSpec((BR, _D), row),
            pl.BlockSpec((_D, _D), full),
            pl.BlockSpec((_D, _D), full),
            pl.BlockSpec((1, _D), full),
        ],
        out_specs=pl.BlockSpec((BR, _D), row),
        out_shape=jax.ShapeDtypeStruct((_N, _D), jnp.float32),
    )(a0, a1, c0, c1, xr, wlT, wrT, b)


def kernel(x, edge_index, W1_l, b1_l, W1_r, W2_l, b2_l, W2_r):
    src = edge_index[0]
    dst = edge_index[1]
    pad = _EP - _E
    src_fill = jnp.arange(pad, dtype=jnp.int32) % _N
    src_p = jnp.concatenate([src, src_fill]).reshape(_EP // _CH, _CH)
    dummy = _N + (jnp.arange(pad, dtype=jnp.int32) % (_NP - _N))
    dst_p = jnp.concatenate([dst, dummy]).reshape(_EP // _CH, _CH)

    acc1, cnt = _sc_agg(src_p, dst_p, x)
    c0 = cnt[0, :_N].reshape(_N, 1)
    c1 = cnt[1, :_N].reshape(_N, 1)
    h = _combine(acc1[0, :_N], acc1[1, :_N], c0, c1, x,
                 W1_l.T, W1_r.T, b1_l.reshape(1, _D), relu=True)
    acc2, _ = _sc_agg(src_p, dst_p, h)
    out = _combine(acc2[0, :_N], acc2[1, :_N], c0, c1, h,
                   W2_l.T, W2_r.T, b2_l.reshape(1, _D), relu=False)
    return out
